# baseline (speedup 1.0000x reference)
import jax
import jax.numpy as jnp
from jax import lax
from jax.experimental import pallas as pl
from jax.experimental.pallas import tpu as pltpu, tpu_sc as plsc

_CHUNK_ROWS = 1000


def _make_sc_kernel(rows, cols):
    mesh = plsc.VectorSubcoreMesh(core_axis_name="c", subcore_axis_name="s")
    num_workers = mesh.num_cores * mesh.num_subcores
    n_chunks = rows // _CHUNK_ROWS
    chunks_per_worker = (n_chunks + num_workers - 1) // num_workers

    def body(x_hbm, ct_hbm, idx_hbm, out_hbm, buf, ct_v, idx_v16,
             sem, row_sem):
        w = lax.axis_index("s") * mesh.num_cores + lax.axis_index("c")

        def chunk_body(i, carry):
            c = w + i * num_workers

            @pl.when(c < n_chunks)
            def _():
                sl = pl.ds(c * _CHUNK_ROWS, _CHUNK_ROWS)
                pltpu.async_copy(x_hbm.at[sl, :], buf, sem).wait()
                pltpu.async_copy(buf, out_hbm.at[sl, :], sem).wait()

            return carry

        lax.fori_loop(0, chunks_per_worker, chunk_body, 0)

        pltpu.async_copy(idx_hbm, idx_v16.at[pl.ds(0, 1)], row_sem).wait()
        idx = idx_v16[...][0]
        owner = (idx // _CHUNK_ROWS) % num_workers

        @pl.when(w == owner)
        def _():
            pltpu.async_copy(ct_hbm, ct_v, row_sem).wait()
            pltpu.async_copy(ct_v, out_hbm.at[pl.ds(idx, 1), :], row_sem).wait()

    return pl.kernel(
        body,
        out_type=jax.ShapeDtypeStruct((rows, cols), jnp.float32),
        mesh=mesh,
        scratch_types=[
            pltpu.VMEM((_CHUNK_ROWS, cols), jnp.float32),
            pltpu.VMEM((1, cols), jnp.float32),
            pltpu.VMEM((16,), jnp.int32),
            pltpu.SemaphoreType.DMA,
            pltpu.SemaphoreType.DMA,
        ],
    )


def kernel(x, copy_tensor, indices):
    rows, cols = x.shape
    return _make_sc_kernel(rows, cols)(x, copy_tensor, indices)
